# Initial kernel scaffold; baseline (speedup 1.0000x reference)
import jax
import jax.numpy as jnp
from jax.experimental import pallas as pl


def kernel(x_l, edge_attr_l, x_c, edge_attr_c, pro_feat, x_pg, ligand_center, We_l, Win_l, W1_l, W2_l, Wout_l, We_c, Win_c, W1_c, W2_c, Wout_c, emb_seq, Wp, Win_pg, W1_pg, W2_pg, Wout_pg, Wo1, bo1, Wo2, bo2, Wo3, bo3, edge_index_l, batch_l, edge_index_c, batch_c, pro_seq, edge_index_pg, batch_pg):
    raise NotImplementedError("write your pallas kernel here")


# speedup vs baseline: 4.1804x; 4.1804x over previous
import functools

import jax
import jax.numpy as jnp
from jax import lax
from jax.experimental import pallas as pl
from jax.experimental.pallas import tpu as pltpu
from jax.experimental.pallas import tpu_sc as plsc

N = 50000
NPAD = 50176
HALF = 25088
STRIPE = HALF // 16
EC = STRIPE // 4
PB = 3136
E = 800000
EPAD = 802816
SLAB = 25000
CAP = 25600
CAPB = CAP // 128
K = 512
KB = 1024
B = 256

_MESH = dict(core_axis_name="c", subcore_axis_name="s", num_cores=2,
             num_subcores=16)
_SC_PARAMS = pltpu.CompilerParams(needs_layout_passes=False,
                                  use_tc_tiling_on_sc=False)


def _pick_bm(m):
    for bm in (4096, 3136, 2048, 1024, 512, 256, 128, 64, 32):
        if m % bm == 0:
            return bm
    return m


def _mm(x, w, relu=False):
    m, k = x.shape
    n = w.shape[1]
    bm = _pick_bm(m)

    def body(x_ref, w_ref, o_ref):
        acc = jnp.dot(x_ref[...], w_ref[...], preferred_element_type=jnp.float32)
        if relu:
            acc = jnp.maximum(acc, 0.0)
        o_ref[...] = acc

    return pl.pallas_call(
        body,
        grid=(m // bm,),
        in_specs=[pl.BlockSpec((bm, k), lambda i: (i, 0)),
                  pl.BlockSpec((k, n), lambda i: (0, 0))],
        out_specs=pl.BlockSpec((bm, n), lambda i: (i, 0)),
        out_shape=jax.ShapeDtypeStruct((m, n), jnp.float32),
    )(x, w)


def _pool(h, batch3d, wout):
    do = wout.shape[1]

    def body(h_ref, b_ref, w_ref, o_ref, s_ref, c_ref):
        i = pl.program_id(0)

        @pl.when(i == 0)
        def _():
            s_ref[...] = jnp.zeros_like(s_ref)
            c_ref[...] = jnp.zeros_like(c_ref)

        ids = b_ref[0, 0, :]
        oh = (lax.broadcasted_iota(jnp.int32, (B, PB), 0)
              == ids[None, :]).astype(jnp.float32)
        s_ref[...] += jnp.dot(oh, h_ref[...], preferred_element_type=jnp.float32)
        c_ref[...] += jnp.sum(oh, axis=1, keepdims=True)

        @pl.when(i == 15)
        def _():
            pooled = s_ref[...] / jnp.maximum(c_ref[...], 1.0)
            o_ref[...] = jnp.dot(pooled, w_ref[...],
                                 preferred_element_type=jnp.float32)

    return pl.pallas_call(
        body,
        grid=(16,),
        in_specs=[pl.BlockSpec((PB, 64), lambda i: (i, 0)),
                  pl.BlockSpec((1, 1, PB), lambda i: (i, 0, 0)),
                  pl.BlockSpec((64, do), lambda i: (0, 0))],
        out_specs=pl.BlockSpec((B, do), lambda i: (0, 0)),
        out_shape=jax.ShapeDtypeStruct((B, do), jnp.float32),
        scratch_shapes=[pltpu.VMEM((B, 64), jnp.float32),
                        pltpu.VMEM((B, 1), jnp.float32)],
    )(h, batch3d, wout)


def _seq_emb(pro_seq, emb_seq_pad):
    def body(s_ref, e_ref, o_ref):
        ids = s_ref[...]
        cols = [jnp.sum((ids == v).astype(jnp.float32), axis=1, keepdims=True)
                for v in range(32)]
        hist = jnp.concatenate(cols, axis=1)
        o_ref[...] = jnp.dot(hist, e_ref[...],
                             preferred_element_type=jnp.float32) * (1.0 / 512.0)

    return pl.pallas_call(
        body,
        out_shape=jax.ShapeDtypeStruct((B, 128), jnp.float32),
    )(pro_seq, emb_seq_pad)


def _pfeat(pf, lc_pad, wp1, wp2):
    def body(a_ref, b_ref, w1_ref, w2_ref, o_ref):
        acc = jnp.dot(a_ref[...], w1_ref[...], preferred_element_type=jnp.float32)
        acc += jnp.dot(b_ref[...], w2_ref[...], preferred_element_type=jnp.float32)
        o_ref[...] = jnp.maximum(acc, 0.0)

    return pl.pallas_call(
        body,
        out_shape=jax.ShapeDtypeStruct((B, 128), jnp.float32),
    )(pf, lc_pad, wp1, wp2)


def _mlp(emb, w1, b1, w2, b2, w3, b3):
    def body(e_ref, w1r, b1r, w2r, b2r, w3r, b3r, o_ref):
        y = jnp.maximum(jnp.dot(e_ref[...], w1r[...],
                                preferred_element_type=jnp.float32) + b1r[...], 0.0)
        y = jnp.maximum(jnp.dot(y, w2r[...],
                                preferred_element_type=jnp.float32) + b2r[...], 0.0)
        y = jnp.maximum(jnp.dot(y, w3r[...],
                                preferred_element_type=jnp.float32) + b3r[...], 0.0)
        o_ref[...] = y

    return pl.pallas_call(
        body,
        out_shape=jax.ShapeDtypeStruct((B, 8), jnp.float32),
    )(emb, w1, b1, w2, b2, w3, b3)


def _partition(src, dst):
    mesh = plsc.VectorSubcoreMesh(**_MESH)

    @functools.partial(
        pl.kernel,
        out_type=[jax.ShapeDtypeStruct((2, 32, CAP), jnp.int32),
                  jax.ShapeDtypeStruct((2, 32, CAP), jnp.int32),
                  jax.ShapeDtypeStruct((2, 32, CAP), jnp.int32),
                  jax.ShapeDtypeStruct((64, 16), jnp.int32)],
        mesh=mesh,
        compiler_params=_SC_PARAMS,
        scratch_types=[pltpu.VMEM((CAP,), jnp.int32),
                       pltpu.VMEM((CAP,), jnp.int32),
                       pltpu.VMEM((CAP,), jnp.int32),
                       pltpu.VMEM((1024,), jnp.int32),
                       pltpu.VMEM((1024,), jnp.int32),
                       pltpu.VMEM((16,), jnp.int32)],
    )
    def kern(src_h, dst_h, psrc_h, pdst_h, peid_h, cnt_h,
             psrc, pdst, peid, sbuf, dbuf, cv):
        wid = lax.axis_index("s") * 2 + lax.axis_index("c")
        iot = lax.broadcasted_iota(jnp.int32, (16,), 0)
        base = wid * SLAB
        for hh in range(2):
            def fill(r, _):
                psrc[pl.ds(r * 16, 16)] = N + iot
                pdst[pl.ds(r * 16, 16)] = (HALF - 32) + iot
                peid[pl.ds(r * 16, 16)] = E + iot
                return 0
            lax.fori_loop(0, CAP // 16, fill, 0)

            def chunk(c, off):
                pltpu.sync_copy(src_h.at[pl.ds(base + c * 1024, 1024)], sbuf)
                pltpu.sync_copy(dst_h.at[pl.ds(base + c * 1024, 1024)], dbuf)

                def body(j, off):
                    s16 = sbuf[pl.ds(j * 16, 16)]
                    d16 = dbuf[pl.ds(j * 16, 16)]
                    el = c * 1024 + j * 16 + iot
                    inb = el < SLAB
                    if hh == 0:
                        inh = d16 < HALF
                    else:
                        inh = d16 >= HALF
                    m = jnp.logical_and(inb, inh)
                    mi = m.astype(jnp.int32)
                    pos = plsc.cumsum(mi) - 1 + off
                    plsc.store_scatter(psrc, [pos], s16, mask=m)
                    plsc.store_scatter(pdst, [pos], d16 - (hh * HALF), mask=m)
                    plsc.store_scatter(peid, [pos], base + el, mask=m)
                    return off + jnp.sum(mi)

                return lax.fori_loop(0, 64, body, off)

            off = lax.fori_loop(0, 25, chunk, jnp.int32(0))
            pltpu.sync_copy(psrc, psrc_h.at[hh, wid])
            pltpu.sync_copy(pdst, pdst_h.at[hh, wid])
            pltpu.sync_copy(peid, peid_h.at[hh, wid])
            cv[...] = jnp.full((16,), off, jnp.int32)
            pltpu.sync_copy(cv, cnt_h.at[hh * 32 + wid])

    return kern(src, dst)


def _sc_layer(hw1, hw2, e, psrc, pdst, peid, cnt, has_e):
    mesh = plsc.VectorSubcoreMesh(**_MESH)
    scratch = [pltpu.VMEM((8, 128), jnp.int32),
               pltpu.VMEM((8, 128), jnp.int32),
               pltpu.VMEM((8, 128), jnp.int32),
               pltpu.VMEM((128, 64), jnp.float32),
               pltpu.VMEM((128, 64), jnp.float32),
               pltpu.VMEM((16,), jnp.int32),
               pltpu.VMEM_SHARED((HALF, 64), jnp.float32),
               pltpu.SemaphoreType.DMA,
               pltpu.SemaphoreType.DMA]
    pieces = [(o, 128) for o in range(0, 1536, 128)] + [(1536, 32)]

    @functools.partial(
        pl.kernel,
        out_type=jax.ShapeDtypeStruct((NPAD, 64), jnp.float32),
        mesh=mesh,
        compiler_params=_SC_PARAMS,
        scratch_types=scratch,
    )
    def kern(hw1_h, hw2_h, e_h, psrc_h, pdst_h, peid_h, cnt_h, hnew_h,
             sidx, didx, eidx, hwrows, erows, cv, agg, sem1, sem2):
        half = lax.axis_index("c")
        tid = lax.axis_index("s")

        def zrow(r, _):
            for q in range(4):
                hwrows[r, pl.ds(q * 16, 16)] = jnp.zeros((16,), jnp.float32)
            return 0
        lax.fori_loop(0, 128, zrow, 0)
        for off, sz in pieces:
            pltpu.sync_copy(hwrows.at[pl.ds(0, sz)],
                            agg.at[pl.ds(tid * STRIPE + off, sz)])
        plsc.subcore_barrier()

        for sl2 in range(2):
            sl = tid * 2 + sl2
            pltpu.sync_copy(cnt_h.at[half * 32 + sl], cv)
            cvv = cv[...]
            trips = (cvv[0] + (KB - 1)) // KB

            def chunk(kk, _):
                pltpu.sync_copy(psrc_h.at[half, sl, pl.ds(kk * 8, 8)], sidx)
                pltpu.sync_copy(pdst_h.at[half, sl, pl.ds(kk * 8, 8)], didx)
                if has_e:
                    pltpu.sync_copy(peid_h.at[half, sl, pl.ds(kk * 8, 8)], eidx)
                for j in range(8):
                    cp1 = pltpu.async_copy(hw1_h.at[sidx.at[j]], hwrows, sem1)
                    if has_e:
                        cp2 = pltpu.async_copy(e_h.at[eidx.at[j]], erows, sem2)
                    cp1.wait()
                    if has_e:
                        cp2.wait()

                        def comp(r, _):
                            for q in range(4):
                                v = hwrows[r, pl.ds(q * 16, 16)]
                                v = v + erows[r, pl.ds(q * 16, 16)]
                                hwrows[r, pl.ds(q * 16, 16)] = jnp.maximum(v, 0.0)
                            return 0
                        lax.fori_loop(0, 128, comp, 0)
                    pltpu.sync_copy(hwrows, agg.at[didx.at[j]], add=True)
                return 0

            lax.fori_loop(0, trips, chunk, 0)

        plsc.subcore_barrier()
        gbase = half * HALF + tid * STRIPE
        for off, sz in pieces:
            pltpu.sync_copy(agg.at[pl.ds(tid * STRIPE + off, sz)],
                            hwrows.at[pl.ds(0, sz)])
            pltpu.sync_copy(hw2_h.at[pl.ds(gbase + off, sz)],
                            erows.at[pl.ds(0, sz)])

            def ep(r, _):
                for q in range(4):
                    v = hwrows[r, pl.ds(q * 16, 16)] + erows[r, pl.ds(q * 16, 16)]
                    hwrows[r, pl.ds(q * 16, 16)] = jnp.maximum(v, 0.0)
                return 0
            lax.fori_loop(0, sz, ep, 0)
            pltpu.sync_copy(hwrows.at[pl.ds(0, sz)],
                            hnew_h.at[pl.ds(gbase + off, sz)])

    return kern(hw1, hw2, e, psrc, pdst, peid, cnt)


def _pad2(a, r, c):
    return jnp.pad(a, ((0, r - a.shape[0]), (0, c - a.shape[1])))


def _gnn_graph(x_pad, win, e, src, dst, w1s, w2s, has_e):
    h = _mm(x_pad, win, relu=True)
    psrc, pdst, peid, cnt = _partition(src, dst)
    rs = lambda a: a.reshape(2, 32, CAPB, 128)
    psrc, pdst, peid = rs(psrc), rs(pdst), rs(peid)
    nlayers = w1s.shape[0]
    for i in range(nlayers):
        hw1 = _mm(h, w1s[i], relu=not has_e)
        hw2 = _mm(h, w2s[i])
        h = _sc_layer(hw1, hw2, e, psrc, pdst, peid, cnt, has_e)
    return h


def kernel(x_l, edge_attr_l, x_c, edge_attr_c, pro_feat, x_pg, ligand_center,
           We_l, Win_l, W1_l, W2_l, Wout_l,
           We_c, Win_c, W1_c, W2_c, Wout_c,
           emb_seq, Wp, Win_pg, W1_pg, W2_pg, Wout_pg,
           Wo1, bo1, Wo2, bo2, Wo3, bo3,
           edge_index_l, batch_l, edge_index_c, batch_c, pro_seq,
           edge_index_pg, batch_pg):
    f32 = jnp.float32

    def pad_edges(ei):
        src = jnp.pad(ei[0], (0, EPAD - E))
        dst = jnp.pad(ei[1], (0, EPAD - E))
        return src.astype(jnp.int32), dst.astype(jnp.int32)

    def pad_batch(b):
        return jnp.pad(b, (0, NPAD - N), constant_values=B).astype(
            jnp.int32).reshape(16, 1, PB)

    src_l, dst_l = pad_edges(edge_index_l)
    e_l = _mm(_pad2(edge_attr_l, EPAD, 16), _pad2(We_l, 16, 64), relu=True)
    h_l = _gnn_graph(_pad2(x_l, NPAD, 32), _pad2(Win_l, 32, 64),
                     e_l, src_l, dst_l, W1_l, W2_l, True)
    l_emb = _pool(h_l, pad_batch(batch_l), Wout_l)

    src_c, dst_c = pad_edges(edge_index_c)
    e_c = _mm(_pad2(edge_attr_c, EPAD, 16), _pad2(We_c, 16, 64), relu=True)
    h_c = _gnn_graph(_pad2(x_c, NPAD, 64), Win_c,
                     e_c, src_c, dst_c, W1_c, W2_c, True)
    c_emb = _pool(h_c, pad_batch(batch_c), Wout_c)

    src_pg, dst_pg = pad_edges(edge_index_pg)
    h_pg = _gnn_graph(_pad2(x_pg, NPAD, 64), Win_pg,
                      e_l, src_pg, dst_pg, W1_pg, W2_pg, False)
    pg_emb = _pool(h_pg, pad_batch(batch_pg), Wout_pg)

    seq = _seq_emb(pro_seq.astype(jnp.int32), _pad2(emb_seq, 32, 128))
    pf = _pfeat(pro_feat, _pad2(ligand_center, B, 8),
                Wp[:1024], _pad2(Wp[1024:], 8, 128))

    emb = jnp.concatenate([l_emb, c_emb, seq, pf, pg_emb], axis=1)
    emb = jnp.pad(emb, ((0, 0), (0, 512 - 432)))
    y = _mlp(emb, _pad2(Wo1, 512, 256), bo1.reshape(1, B).astype(f32),
             Wo2, bo2.reshape(1, 128), _pad2(Wo3, 128, 8),
             jnp.pad(bo3, (0, 7)).reshape(1, 8))
    return y[:, 0]
